# Initial kernel scaffold; baseline (speedup 1.0000x reference)
import functools

import jax
import jax.numpy as jnp
from jax import lax
from jax.experimental import pallas as pl
from jax.experimental.pallas import tpu as pltpu
from jax.experimental.pallas import tpu_sc as plsc

NUM_EMB = 100000
DIM = 64
BATCH = 1024
SEQ = 200
TOT = BATCH * SEQ

NUM_CORES = 2
NUM_SUBCORES = 16
NW = NUM_CORES * NUM_SUBCORES
ROWS_PER_W = TOT // NW
CHUNK = 128
NCHUNK = ROWS_PER_W // CHUNK


def _make_kernel():
    mesh = plsc.VectorSubcoreMesh(core_axis_name="c", subcore_axis_name="s")

    @functools.partial(
        pl.kernel,
        mesh=mesh,
        out_type=jax.ShapeDtypeStruct((NW, NCHUNK, CHUNK, DIM), jnp.float32),
        scratch_types=[
            pltpu.VMEM((NCHUNK, CHUNK), jnp.int32),
            pltpu.VMEM((CHUNK, DIM), jnp.float32),
            pltpu.SemaphoreType.DMA,
        ],
    )
    def k(idx_hbm, table_hbm, out_hbm, idx_v, rows_v, sem):
        wid = lax.axis_index("s") * NUM_CORES + lax.axis_index("c")
        pltpu.sync_copy(idx_hbm.at[wid], idx_v)

        def body(j, carry):
            pltpu.async_copy(table_hbm.at[idx_v.at[j]], rows_v, sem).wait()
            pltpu.sync_copy(rows_v, out_hbm.at[wid, j])
            return carry

        lax.fori_loop(0, NCHUNK, body, 0)

    return k


_gather_kernel = _make_kernel()


def kernel(x, weight):
    xr = x.reshape(NW, NCHUNK, CHUNK)
    out = _gather_kernel(xr, weight)
    return out.reshape(BATCH, SEQ, DIM)


# speedup vs baseline: 2.8778x; 2.8778x over previous
import functools

import jax
import jax.numpy as jnp
from jax import lax
from jax.experimental import pallas as pl
from jax.experimental.pallas import tpu as pltpu
from jax.experimental.pallas import tpu_sc as plsc

NUM_EMB = 100000
DIM = 64
BATCH = 1024
SEQ = 200
TOT = BATCH * SEQ

NUM_CORES = 2
NUM_SUBCORES = 16
NW = NUM_CORES * NUM_SUBCORES
ROWS_PER_W = TOT // NW
CHUNK = 128
NCHUNK = ROWS_PER_W // CHUNK


def _make_kernel():
    mesh = plsc.VectorSubcoreMesh(core_axis_name="c", subcore_axis_name="s")

    @functools.partial(
        pl.kernel,
        mesh=mesh,
        out_type=jax.ShapeDtypeStruct((NW, NCHUNK, CHUNK, DIM), jnp.float32),
        scratch_types=[
            pltpu.VMEM((NCHUNK, CHUNK), jnp.int32),
            pltpu.VMEM((CHUNK, DIM), jnp.float32),
            pltpu.SemaphoreType.DMA,
        ],
        compiler_params=pltpu.CompilerParams(use_tc_tiling_on_sc=False),
    )
    def k(idx_hbm, table_hbm, out_hbm, idx_v, rows_v, sem):
        wid = lax.axis_index("s") * NUM_CORES + lax.axis_index("c")
        pltpu.sync_copy(idx_hbm.at[wid], idx_v)

        def body(j, carry):
            pltpu.async_copy(table_hbm.at[idx_v.at[j]], rows_v, sem).wait()
            pltpu.sync_copy(rows_v, out_hbm.at[wid, j])
            return carry

        lax.fori_loop(0, NCHUNK, body, 0)

    return k


_gather_kernel = _make_kernel()


def kernel(x, weight):
    xr = x.reshape(NW, NCHUNK, CHUNK)
    out = _gather_kernel(xr, weight)
    return out.reshape(BATCH, SEQ, DIM)


# speedup vs baseline: 3.3076x; 1.1494x over previous
import functools

import jax
import jax.numpy as jnp
from jax import lax
from jax.experimental import pallas as pl
from jax.experimental.pallas import tpu as pltpu
from jax.experimental.pallas import tpu_sc as plsc

NUM_EMB = 100000
DIM = 64
BATCH = 1024
SEQ = 200
TOT = BATCH * SEQ

NUM_CORES = 2
NUM_SUBCORES = 16
NW = NUM_CORES * NUM_SUBCORES
ROWS_PER_W = TOT // NW
CHUNK = 128
NCHUNK = ROWS_PER_W // CHUNK
NBUF = 5


def _make_kernel():
    mesh = plsc.VectorSubcoreMesh(core_axis_name="c", subcore_axis_name="s")

    @functools.partial(
        pl.kernel,
        mesh=mesh,
        out_type=jax.ShapeDtypeStruct((NW, NCHUNK, CHUNK, DIM), jnp.float32),
        scratch_types=[
            pltpu.VMEM((NCHUNK, CHUNK), jnp.int32),
            pltpu.VMEM((NBUF, CHUNK, DIM), jnp.float32),
        ]
        + [pltpu.SemaphoreType.DMA] * (2 * NBUF),
        compiler_params=pltpu.CompilerParams(use_tc_tiling_on_sc=False),
    )
    def k(idx_hbm, table_hbm, out_hbm, idx_v, rows_v, *sems):
        sem_g = sems[:NBUF]
        sem_w = sems[NBUF:]
        wid = lax.axis_index("s") * NUM_CORES + lax.axis_index("c")
        pltpu.sync_copy(idx_hbm.at[wid], idx_v)

        def gather_start(j, b):
            pltpu.async_copy(table_hbm.at[idx_v.at[j]], rows_v.at[b], sem_g[b])

        def writeback_start(j, b):
            pltpu.async_copy(rows_v.at[b], out_hbm.at[wid, j], sem_w[b])

        for b in range(NBUF):
            gather_start(b, b)

        def round_body(i, carry):
            j0 = i * NBUF
            for b in range(NBUF):
                j = j0 + b
                pltpu.make_async_copy(table_hbm.at[idx_v.at[j]], rows_v.at[b], sem_g[b]).wait()
                writeback_start(j, b)
                pltpu.make_async_copy(rows_v.at[b], out_hbm.at[wid, j], sem_w[b]).wait()
                gather_start(j + NBUF, b)
            return carry

        lax.fori_loop(0, NCHUNK // NBUF - 1, round_body, 0)

        j0 = NCHUNK - NBUF
        for b in range(NBUF):
            j = j0 + b
            pltpu.make_async_copy(table_hbm.at[idx_v.at[j]], rows_v.at[b], sem_g[b]).wait()
            writeback_start(j, b)
        for b in range(NBUF):
            j = j0 + b
            pltpu.make_async_copy(rows_v.at[b], out_hbm.at[wid, j], sem_w[b]).wait()

    return k


_gather_kernel = _make_kernel()


def kernel(x, weight):
    xr = x.reshape(NW, NCHUNK, CHUNK)
    out = _gather_kernel(xr, weight)
    return out.reshape(BATCH, SEQ, DIM)


# speedup vs baseline: 3.4899x; 1.0551x over previous
import functools

import jax
import jax.numpy as jnp
from jax import lax
from jax.experimental import pallas as pl
from jax.experimental.pallas import tpu as pltpu
from jax.experimental.pallas import tpu_sc as plsc

NUM_EMB = 100000
DIM = 64
BATCH = 1024
SEQ = 200

NUM_CORES = 2
NUM_SUBCORES = 16
NW = NUM_CORES * NUM_SUBCORES
NPASS = DIM // NW


def _make_kernel():
    mesh = plsc.VectorSubcoreMesh(core_axis_name="c", subcore_axis_name="s")

    @functools.partial(
        pl.kernel,
        mesh=mesh,
        out_type=jax.ShapeDtypeStruct((SEQ, DIM, BATCH), jnp.float32),
        scratch_types=[
            pltpu.VMEM((NUM_EMB,), jnp.float32),
            pltpu.VMEM((2, 8, 128), jnp.int32),
            pltpu.VMEM((2, BATCH), jnp.float32),
        ]
        + [pltpu.SemaphoreType.DMA] * 4,
        compiler_params=pltpu.CompilerParams(needs_layout_passes=False),
    )
    def k(idx_hbm, wt_hbm, out_hbm, table_v, idx_v, row_v, si0, si1, so0, so1):
        sem_i = (si0, si1)
        sem_o = (so0, so1)
        wid = lax.axis_index("s") * NUM_CORES + lax.axis_index("c")

        for p in range(NPASS):
            d = p * NW + wid
            pltpu.sync_copy(wt_hbm.at[d], table_v)
            pltpu.async_copy(idx_hbm.at[0], idx_v.at[0], sem_i[0])
            pltpu.async_copy(idx_hbm.at[1], idx_v.at[1], sem_i[1])

            def sbody(i, carry, p=p, d=d):
                for b in range(2):
                    s = 2 * i + b
                    pltpu.make_async_copy(
                        idx_hbm.at[s], idx_v.at[b], sem_i[b]
                    ).wait()

                    @pl.when(i > 0)
                    def _():
                        pltpu.make_async_copy(
                            row_v.at[b], out_hbm.at[s, d], sem_o[b]
                        ).wait()

                    for g in range(BATCH // 16):
                        ivec = idx_v[b, g // 8, pl.ds((g % 8) * 16, 16)]
                        vals = plsc.load_gather(table_v, [ivec])
                        row_v[b, pl.ds(g * 16, 16)] = vals

                    pltpu.async_copy(row_v.at[b], out_hbm.at[s, d], sem_o[b])

                    @pl.when(i < SEQ // 2 - 1)
                    def _():
                        pltpu.async_copy(idx_hbm.at[s + 2], idx_v.at[b], sem_i[b])
                return carry

            lax.fori_loop(0, SEQ // 2, sbody, 0)
            for b in range(2):
                pltpu.make_async_copy(
                    row_v.at[b], out_hbm.at[b, d], sem_o[b]
                ).wait()

    return k


_gather_kernel = _make_kernel()


def kernel(x, weight):
    x2 = x.T.reshape(SEQ, 8, 128)
    out = _gather_kernel(x2, weight.T)
    return out.transpose(2, 0, 1)


# speedup vs baseline: 4.1053x; 1.1763x over previous
import functools

import jax
import jax.numpy as jnp
from jax import lax
from jax.experimental import pallas as pl
from jax.experimental.pallas import tpu as pltpu
from jax.experimental.pallas import tpu_sc as plsc

NUM_EMB = 100000
DIM = 64
BATCH = 1024
SEQ = 200

NUM_CORES = 2
NUM_SUBCORES = 16
NW = NUM_CORES * NUM_SUBCORES
NPASS = DIM // NW


def _make_kernel():
    mesh = plsc.VectorSubcoreMesh(core_axis_name="c", subcore_axis_name="s")

    @functools.partial(
        pl.kernel,
        mesh=mesh,
        out_type=jax.ShapeDtypeStruct((SEQ, DIM, BATCH), jnp.float32),
        scratch_types=[
            pltpu.VMEM((NUM_EMB,), jnp.float32),
            pltpu.VMEM((2, 8, 128), jnp.int32),
            pltpu.VMEM((2, BATCH), jnp.float32),
        ]
        + [pltpu.SemaphoreType.DMA] * 4,
        compiler_params=pltpu.CompilerParams(needs_layout_passes=False),
    )
    def k(idx_hbm, wt_hbm, out_hbm, table_v, idx_v, row_v, si0, si1, so0, so1):
        sem_i = (si0, si1)
        sem_o = (so0, so1)
        wid = lax.axis_index("s") * NUM_CORES + lax.axis_index("c")

        for p in range(NPASS):
            d = p * NW + wid
            pltpu.sync_copy(wt_hbm.at[d], table_v)
            pltpu.async_copy(idx_hbm.at[0], idx_v.at[0], sem_i[0])
            pltpu.async_copy(idx_hbm.at[1], idx_v.at[1], sem_i[1])

            def sbody(i, carry, p=p, d=d):
                for b in range(2):
                    s = 2 * i + b
                    pltpu.make_async_copy(
                        idx_hbm.at[s], idx_v.at[b], sem_i[b]
                    ).wait()

                    @pl.when(i > 0)
                    def _():
                        pltpu.make_async_copy(
                            row_v.at[b], out_hbm.at[s, d], sem_o[b]
                        ).wait()

                    @plsc.parallel_loop(0, BATCH // 16, unroll=8)
                    def _(g, b=b):
                        ivec = idx_v[b, g // 8, pl.ds((g % 8) * 16, 16)]
                        vals = plsc.load_gather(table_v, [ivec])
                        row_v[b, pl.ds(g * 16, 16)] = vals

                    pltpu.async_copy(row_v.at[b], out_hbm.at[s, d], sem_o[b])

                    @pl.when(i < SEQ // 2 - 1)
                    def _():
                        pltpu.async_copy(idx_hbm.at[s + 2], idx_v.at[b], sem_i[b])
                return carry

            lax.fori_loop(0, SEQ // 2, sbody, 0)
            for b in range(2):
                pltpu.make_async_copy(
                    row_v.at[b], out_hbm.at[b, d], sem_o[b]
                ).wait()

    return k


_gather_kernel = _make_kernel()


def kernel(x, weight):
    x2 = x.T.reshape(SEQ, 8, 128)
    out = _gather_kernel(x2, weight.T)
    return out.transpose(2, 0, 1)


# speedup vs baseline: 6.5921x; 1.6057x over previous
import functools

import jax
import jax.numpy as jnp
from jax import lax
from jax.experimental import pallas as pl
from jax.experimental.pallas import tpu as pltpu
from jax.experimental.pallas import tpu_sc as plsc

NUM_EMB = 100000
DIM = 64
BATCH = 1024
SEQ = 200

NUM_CORES = 2
NUM_SUBCORES = 16
NW = NUM_CORES * NUM_SUBCORES
NPASS = DIM // NW
GRP = 4
RNB = 4


def _make_kernel():
    mesh = plsc.VectorSubcoreMesh(core_axis_name="c", subcore_axis_name="s")

    @functools.partial(
        pl.kernel,
        mesh=mesh,
        out_type=jax.ShapeDtypeStruct((SEQ, DIM, BATCH), jnp.float32),
        scratch_types=[
            pltpu.VMEM((NUM_EMB,), jnp.float32),
            pltpu.VMEM((2, GRP, 8, 128), jnp.int32),
            pltpu.VMEM((RNB, BATCH), jnp.float32),
        ]
        + [pltpu.SemaphoreType.DMA] * (2 + RNB),
        compiler_params=pltpu.CompilerParams(needs_layout_passes=False),
    )
    def k(idx_hbm, wt_hbm, out_hbm, table_v, idx_v, row_v, *sems):
        sem_i = sems[:2]
        sem_o = sems[2:]
        wid = lax.axis_index("s") * NUM_CORES + lax.axis_index("c")
        ngrp = SEQ // GRP

        for p in range(NPASS):
            d = p * NW + wid
            pltpu.sync_copy(wt_hbm.at[d], table_v)
            pltpu.async_copy(idx_hbm.at[pl.ds(0, GRP)], idx_v.at[0], sem_i[0])
            pltpu.async_copy(idx_hbm.at[pl.ds(GRP, GRP)], idx_v.at[1], sem_i[1])

            def gbody(i, carry, p=p, d=d):
                for gb in range(2):
                    g = 2 * i + gb
                    s0 = g * GRP
                    pltpu.make_async_copy(
                        idx_hbm.at[pl.ds(s0, GRP)], idx_v.at[gb], sem_i[gb]
                    ).wait()
                    for j in range(GRP):
                        s = s0 + j
                        b = j % RNB

                        @pl.when(g * GRP + j >= RNB)
                        def _():
                            pltpu.make_async_copy(
                                row_v.at[b], out_hbm.at[s, d], sem_o[b]
                            ).wait()

                        @plsc.parallel_loop(0, BATCH // 16, unroll=8)
                        def _(q, gb=gb, j=j, b=b):
                            ivec = idx_v[gb, j, q // 8, pl.ds((q % 8) * 16, 16)]
                            vals = plsc.load_gather(table_v, [ivec])
                            row_v[b, pl.ds(q * 16, 16)] = vals

                        pltpu.async_copy(row_v.at[b], out_hbm.at[s, d], sem_o[b])

                    @pl.when(g < ngrp - 2)
                    def _(gb=gb, s0=s0):
                        pltpu.async_copy(
                            idx_hbm.at[pl.ds(s0 + 2 * GRP, GRP)],
                            idx_v.at[gb],
                            sem_i[gb],
                        )
                return carry

            lax.fori_loop(0, ngrp // 2, gbody, 0)
            for b in range(RNB):
                pltpu.make_async_copy(
                    row_v.at[b], out_hbm.at[b, d], sem_o[b]
                ).wait()

    return k


_gather_kernel = _make_kernel()


def kernel(x, weight):
    x2 = x.T.reshape(SEQ, 8, 128)
    out = _gather_kernel(x2, weight.T)
    return out.transpose(2, 0, 1)


# speedup vs baseline: 6.6376x; 1.0069x over previous
import functools

import jax
import jax.numpy as jnp
from jax import lax
from jax.experimental import pallas as pl
from jax.experimental.pallas import tpu as pltpu
from jax.experimental.pallas import tpu_sc as plsc

NUM_EMB = 100000
DIM = 64
BATCH = 1024
SEQ = 200

NUM_CORES = 2
NUM_SUBCORES = 16
NW = NUM_CORES * NUM_SUBCORES
NPASS = DIM // NW
GRP = 4
RNB = 8


def _make_kernel():
    mesh = plsc.VectorSubcoreMesh(core_axis_name="c", subcore_axis_name="s")

    @functools.partial(
        pl.kernel,
        mesh=mesh,
        out_type=jax.ShapeDtypeStruct((SEQ, DIM, BATCH), jnp.float32),
        scratch_types=[
            pltpu.VMEM((NUM_EMB,), jnp.float32),
            pltpu.VMEM((2, GRP, 8, 128), jnp.int32),
            pltpu.VMEM((RNB, BATCH), jnp.float32),
        ]
        + [pltpu.SemaphoreType.DMA] * (2 + RNB),
        compiler_params=pltpu.CompilerParams(needs_layout_passes=False),
    )
    def k(idx_hbm, wt_hbm, out_hbm, table_v, idx_v, row_v, *sems):
        sem_i = sems[:2]
        sem_o = sems[2:]
        wid = lax.axis_index("s") * NUM_CORES + lax.axis_index("c")
        ngrp = SEQ // GRP

        for p in range(NPASS):
            d = p * NW + wid
            pltpu.async_copy(idx_hbm.at[pl.ds(0, GRP)], idx_v.at[0], sem_i[0])
            pltpu.async_copy(idx_hbm.at[pl.ds(GRP, GRP)], idx_v.at[1], sem_i[1])
            pltpu.sync_copy(wt_hbm.at[d], table_v)

            def gbody(i, carry, p=p, d=d):
                for gb in range(2):
                    g = 2 * i + gb
                    s0 = g * GRP
                    pltpu.make_async_copy(
                        idx_hbm.at[pl.ds(s0, GRP)], idx_v.at[gb], sem_i[gb]
                    ).wait()
                    for j in range(GRP):
                        s = s0 + j
                        b = (gb * GRP + j) % RNB

                        @pl.when(g * GRP + j >= RNB)
                        def _():
                            pltpu.make_async_copy(
                                row_v.at[b], out_hbm.at[s, d], sem_o[b]
                            ).wait()

                        @plsc.parallel_loop(0, BATCH // 16, unroll=8)
                        def _(q, gb=gb, j=j, b=b):
                            ivec = idx_v[gb, j, q // 8, pl.ds((q % 8) * 16, 16)]
                            vals = plsc.load_gather(table_v, [ivec])
                            row_v[b, pl.ds(q * 16, 16)] = vals

                        pltpu.async_copy(row_v.at[b], out_hbm.at[s, d], sem_o[b])

                    @pl.when(g < ngrp - 2)
                    def _(gb=gb, s0=s0):
                        pltpu.async_copy(
                            idx_hbm.at[pl.ds(s0 + 2 * GRP, GRP)],
                            idx_v.at[gb],
                            sem_i[gb],
                        )
                return carry

            lax.fori_loop(0, ngrp // 2, gbody, 0)
            for b in range(RNB):
                pltpu.make_async_copy(
                    row_v.at[b], out_hbm.at[b, d], sem_o[b]
                ).wait()

    return k


_gather_kernel = _make_kernel()


def kernel(x, weight):
    x2 = x.T.reshape(SEQ, 8, 128)
    out = _gather_kernel(x2, weight.T)
    return out.transpose(2, 0, 1)


# speedup vs baseline: 6.7945x; 1.0236x over previous
import functools

import jax
import jax.numpy as jnp
from jax import lax
from jax.experimental import pallas as pl
from jax.experimental.pallas import tpu as pltpu
from jax.experimental.pallas import tpu_sc as plsc

NUM_EMB = 100000
DIM = 64
BATCH = 1024
SEQ = 200

NUM_CORES = 2
NUM_SUBCORES = 16
NW = NUM_CORES * NUM_SUBCORES
NPASS = DIM // NW
GRP = 4
RNB = 2


def _make_kernel():
    mesh = plsc.VectorSubcoreMesh(core_axis_name="c", subcore_axis_name="s")

    @functools.partial(
        pl.kernel,
        mesh=mesh,
        out_type=jax.ShapeDtypeStruct((SEQ, DIM, BATCH), jnp.float32),
        scratch_types=[
            pltpu.VMEM((NUM_EMB,), jnp.float32),
            pltpu.VMEM((2, GRP, 8, 128), jnp.int32),
            pltpu.VMEM((RNB, GRP, BATCH), jnp.float32),
        ]
        + [pltpu.SemaphoreType.DMA] * (2 + RNB),
        compiler_params=pltpu.CompilerParams(needs_layout_passes=False),
    )
    def k(idx_hbm, wt_hbm, out_hbm, table_v, idx_v, row_v, *sems):
        sem_i = sems[:2]
        sem_o = sems[2:]
        wid = lax.axis_index("s") * NUM_CORES + lax.axis_index("c")
        ngrp = SEQ // GRP

        for p in range(NPASS):
            d = p * NW + wid
            pltpu.async_copy(idx_hbm.at[pl.ds(0, GRP)], idx_v.at[0], sem_i[0])
            pltpu.async_copy(idx_hbm.at[pl.ds(GRP, GRP)], idx_v.at[1], sem_i[1])
            pltpu.sync_copy(wt_hbm.at[d], table_v)

            def gbody(i, carry, p=p, d=d):
                for gb in range(2):
                    g = 2 * i + gb
                    s0 = g * GRP
                    b = gb
                    pltpu.make_async_copy(
                        idx_hbm.at[pl.ds(s0, GRP)], idx_v.at[gb], sem_i[gb]
                    ).wait()

                    @pl.when(g >= RNB)
                    def _():
                        pltpu.make_async_copy(
                            row_v.at[b],
                            out_hbm.at[pl.ds(s0, GRP), d],
                            sem_o[b],
                        ).wait()

                    @plsc.parallel_loop(0, GRP * BATCH // 16, unroll=8)
                    def _(q, gb=gb, b=b):
                        ivec = idx_v[
                            gb, q // 64, (q // 8) % 8, pl.ds((q % 8) * 16, 16)
                        ]
                        vals = plsc.load_gather(table_v, [ivec])
                        row_v[b, q // 64, pl.ds((q % 64) * 16, 16)] = vals

                    pltpu.async_copy(
                        row_v.at[b], out_hbm.at[pl.ds(s0, GRP), d], sem_o[b]
                    )

                    @pl.when(g < ngrp - 2)
                    def _(gb=gb, s0=s0):
                        pltpu.async_copy(
                            idx_hbm.at[pl.ds(s0 + 2 * GRP, GRP)],
                            idx_v.at[gb],
                            sem_i[gb],
                        )
                return carry

            lax.fori_loop(0, ngrp // 2, gbody, 0)
            for b in range(RNB):
                pltpu.make_async_copy(
                    row_v.at[b], out_hbm.at[pl.ds(b, GRP), d], sem_o[b]
                ).wait()

    return k


_gather_kernel = _make_kernel()


def kernel(x, weight):
    x2 = x.T.reshape(SEQ, 8, 128)
    out = _gather_kernel(x2, weight.T)
    return out.transpose(2, 0, 1)
